# baseline (speedup 1.0000x reference)
import functools

import jax
import jax.numpy as jnp
from jax import lax
from jax.experimental import pallas as pl
from jax.experimental.pallas import tpu as pltpu
from jax.experimental.pallas import tpu_sc as plsc

N = 10000
E = 320000
D = 128
NS = 100
B = 100
LAT = 5

NUM_CORES = 2
NUM_SUBCORES = 16
NUM_WORKERS = NUM_CORES * NUM_SUBCORES

CHUNK = 125
CPT = 80
TOT_CHUNKS = E // CHUNK
IDX_BLK = 16
NSTAGES = CPT // IDX_BLK
assert NUM_WORKERS * CPT == TOT_CHUNKS

ZBLK = 200
NZB = N // ZBLK
ZB_ITERS = -(-NZB // NUM_SUBCORES)


def _sc_agg_body(x_hbm, z_hbm, ei_hbm, out_hbm,
                 srcblk, dstblk, rows, acc, sem0, sem1):
    cid = lax.axis_index("c")
    sid = lax.axis_index("s")
    wid = cid * NUM_SUBCORES + sid

    def stage_idx(k):
        off = wid * CPT + k * IDX_BLK
        pltpu.sync_copy(ei_hbm.at[0, pl.ds(off, IDX_BLK)], srcblk.at[k % 2])
        pltpu.sync_copy(ei_hbm.at[1, pl.ds(off, IDX_BLK)], dstblk.at[k % 2])

    stage_idx(0)

    def zblock(k, _):
        blk = k * NUM_SUBCORES + sid

        @pl.when(blk < NZB)
        def _():
            pltpu.sync_copy(z_hbm, acc.at[pl.ds(blk * ZBLK, ZBLK)])
        return 0
    lax.fori_loop(0, ZB_ITERS, zblock, 0)
    plsc.subcore_barrier()

    rows0 = rows.at[0]
    rows1 = rows.at[1]
    for k in range(NSTAGES):
        sb = srcblk.at[k % 2]
        db = dstblk.at[k % 2]
        pltpu.async_copy(x_hbm.at[sb.at[0]], rows0, sem0)
        if k + 1 < NSTAGES:
            stage_idx(k + 1)

        def pair_body(j, _):
            c0 = 2 * j
            c1 = 2 * j + 1
            pltpu.async_copy(x_hbm.at[sb.at[c1]], rows1, sem1)
            pltpu.make_async_copy(x_hbm.at[sb.at[c0]], rows0, sem0).wait()
            pltpu.sync_copy(rows0, acc.at[db.at[c0]], add=True)

            @pl.when(c1 + 1 < IDX_BLK)
            def _():
                pltpu.async_copy(
                    x_hbm.at[sb.at[jnp.minimum(c1 + 1, IDX_BLK - 1)]],
                    rows0, sem0)
            pltpu.make_async_copy(x_hbm.at[sb.at[c1]], rows1, sem1).wait()
            pltpu.sync_copy(rows1, acc.at[db.at[c1]], add=True)
            return 0
        lax.fori_loop(0, IDX_BLK // 2, pair_body, 0)

    plsc.subcore_barrier()

    def wblock(k, _):
        blk = k * NUM_SUBCORES + sid

        @pl.when(blk < NZB)
        def _():
            pltpu.sync_copy(acc.at[pl.ds(blk * ZBLK, ZBLK)],
                            out_hbm.at[cid, pl.ds(blk * ZBLK, ZBLK)])
        return 0
    lax.fori_loop(0, ZB_ITERS, wblock, 0)


@functools.cache
def _sc_agg():
    return pl.kernel(
        _sc_agg_body,
        mesh=plsc.VectorSubcoreMesh(
            core_axis_name="c", subcore_axis_name="s",
            num_cores=NUM_CORES, num_subcores=NUM_SUBCORES),
        out_type=jax.ShapeDtypeStruct((NUM_CORES, N, D), jnp.float32),
        scratch_types=[
            pltpu.VMEM((2, IDX_BLK, CHUNK), jnp.int32),
            pltpu.VMEM((2, IDX_BLK, CHUNK), jnp.int32),
            pltpu.VMEM((2, CHUNK, D), jnp.float32),
            pltpu.VMEM_SHARED((N, D), jnp.float32),
            pltpu.SemaphoreType.DMA,
            pltpu.SemaphoreType.DMA,
        ],
    )


ROWS_TC = 1000


def _tc1_body(p_ref, wg_ref, wo_ref, wm_ref, uw_ref, lc_ref, u_ref):
    agg = p_ref[0] + p_ref[1]
    h = jnp.maximum(jnp.dot(agg, wg_ref[...],
                            preferred_element_type=jnp.float32), 0.0)
    od = jnp.dot(h, wo_ref[...], preferred_element_type=jnp.float32)
    q = jnp.dot(od, wm_ref[...], preferred_element_type=jnp.float32)
    col = lax.broadcasted_iota(jnp.int32, (ROWS_TC, NS * LAT), 1)
    jloc = lax.broadcasted_iota(jnp.int32, (ROWS_TC, NS * LAT), 0) % NS
    lc_ref[...] = jnp.concatenate(
        [jnp.sum(jnp.where(col == jloc * LAT + l, q, 0.0), axis=1,
                 keepdims=True) for l in range(LAT)], axis=1)
    u = (jnp.sum(h, axis=1) * (uw_ref[0] / D)
         + jnp.sum(agg, axis=1) * (uw_ref[1] / D))
    u_ref[...] = u.reshape(ROWS_TC, 1)


def _tc1(partials, W_gnn, W_od, W_mat, utility_w):
    grid = N // ROWS_TC
    return pl.pallas_call(
        _tc1_body,
        grid=(grid,),
        in_specs=[
            pl.BlockSpec((NUM_CORES, ROWS_TC, D), lambda i: (0, i, 0)),
            pl.BlockSpec((D, D), lambda i: (0, 0)),
            pl.BlockSpec((D, NS), lambda i: (0, 0)),
            pl.BlockSpec((NS, NS * LAT), lambda i: (0, 0)),
            pl.BlockSpec(memory_space=pltpu.SMEM),
        ],
        out_specs=[
            pl.BlockSpec((ROWS_TC, LAT), lambda i: (i, 0)),
            pl.BlockSpec((ROWS_TC, 1), lambda i: (i, 0)),
        ],
        out_shape=[
            jax.ShapeDtypeStruct((N, LAT), jnp.float32),
            jax.ShapeDtypeStruct((N, 1), jnp.float32),
        ],
    )(partials, W_gnn, W_od, W_mat, utility_w)


def _tc2_body(lc_ref, u_ref, be_ref, wd_ref, bd_ref, out_ref):
    grp = lax.broadcasted_iota(jnp.int32, (B, N), 0)
    row = lax.broadcasted_iota(jnp.int32, (B, N), 1)
    gmat = jnp.where(row // NS == grp, 1.0, 0.0)
    lat = jnp.maximum(
        jnp.dot(gmat, lc_ref[...], preferred_element_type=jnp.float32)
        + be_ref[...], 0.0)
    rec = jnp.dot(lat, wd_ref[...],
                  preferred_element_type=jnp.float32) + bd_ref[...]
    seat = lax.broadcasted_iota(jnp.int32, (N, NS), 1)
    rloc = lax.broadcasted_iota(jnp.int32, (N, NS), 0) % NS
    usel = jnp.where(seat == rloc, u_ref[...], 0.0)
    u2 = jnp.dot(gmat, usel, preferred_element_type=jnp.float32)
    m = jnp.max(u2, axis=1, keepdims=True)
    e = jnp.exp(u2 - m)
    prob = e / jnp.sum(e, axis=1, keepdims=True)
    out_ref[:, :NS] = prob
    out_ref[:, NS:] = rec


def _tc2(lc, u, b_enc, W_dec, b_dec):
    return pl.pallas_call(
        _tc2_body,
        out_shape=jax.ShapeDtypeStruct((B, NS + N), jnp.float32),
    )(lc, u, b_enc.reshape(1, LAT), W_dec, b_dec.reshape(1, NS * NS))


def kernel(x_od, edge_index, W_gnn, W_od, W_enc, b_enc, W_dec, b_dec, utility_w):
    ei = edge_index.reshape(2, TOT_CHUNKS, CHUNK)
    zsrc = jnp.zeros((ZBLK, D), jnp.float32)
    partials = _sc_agg()(x_od, zsrc, ei)
    W_mat = W_enc.reshape(NS, NS, LAT).transpose(1, 0, 2).reshape(NS, NS * LAT)
    lc, u = _tc1(partials, W_gnn, W_od, W_mat, utility_w)
    return _tc2(lc, u, b_enc, W_dec, b_dec)
